# Initial kernel scaffold; baseline (speedup 1.0000x reference)
import jax
import jax.numpy as jnp
from jax.experimental import pallas as pl


def kernel(input, lookup):
    raise NotImplementedError("write your pallas kernel here")


# speedup vs baseline: 1.0938x; 1.0938x over previous
import functools

import jax
import jax.numpy as jnp
from jax import lax
from jax.experimental import pallas as pl
from jax.experimental.pallas import tpu as pltpu
from jax.experimental.pallas import tpu_sc as plsc

_NUM_CORES = 2
_NUM_SUBCORES = 16
_NUM_WORKERS = _NUM_CORES * _NUM_SUBCORES
_IDX_PER_STREAM = 128


@functools.cache
def _make_gather(n_idx: int, vocab: int, dim: int):
  per_worker = n_idx // _NUM_WORKERS
  assert per_worker * _NUM_WORKERS == n_idx
  chunk = 1024
  while per_worker % chunk:
    chunk //= 2
  n_streams = chunk // _IDX_PER_STREAM
  n_chunks = per_worker // chunk

  mesh = plsc.VectorSubcoreMesh(core_axis_name="c", subcore_axis_name="s")

  @functools.partial(
      pl.kernel,
      mesh=mesh,
      out_type=jax.ShapeDtypeStruct((n_idx, dim), jnp.float32),
      scratch_types=[
          pltpu.VMEM((n_streams, _IDX_PER_STREAM), jnp.int32),
          pltpu.VMEM((chunk, dim), jnp.float32),
          pltpu.SemaphoreType.DMA,
      ],
      compiler_params=pltpu.CompilerParams(use_tc_tiling_on_sc=False),
  )
  def gather(idx_hbm, table_hbm, out_hbm, idx_v, rows_v, sem):
    wid = lax.axis_index("s") * _NUM_CORES + lax.axis_index("c")
    base = wid * per_worker

    def body(i, _):
      off = pl.multiple_of(base + i * chunk, chunk)
      row = pl.multiple_of(
          base // _IDX_PER_STREAM + i * n_streams, min(n_streams, 8)
      )
      pltpu.sync_copy(idx_hbm.at[pl.ds(row, n_streams)], idx_v)
      copies = []
      for j in range(n_streams):
        copies.append(
            pltpu.async_copy(
                table_hbm.at[idx_v.at[j]],
                rows_v.at[pl.ds(j * _IDX_PER_STREAM, _IDX_PER_STREAM)],
                sem,
            )
        )
      for c in copies:
        c.wait()
      pltpu.sync_copy(rows_v, out_hbm.at[pl.ds(off, chunk)])
      return 0

    lax.fori_loop(0, n_chunks, body, 0)

  return gather


def kernel(input, lookup):
  batch, hist = input.shape
  vocab, dim = lookup.shape
  n_idx = batch * hist
  idx = input.reshape(n_idx // _IDX_PER_STREAM, _IDX_PER_STREAM).astype(
      jnp.int32
  )
  out = _make_gather(n_idx, vocab, dim)(idx, lookup)
  return out.reshape(batch, hist, dim)


# speedup vs baseline: 1.1103x; 1.0150x over previous
import functools

import jax
import jax.numpy as jnp
from jax import lax
from jax.experimental import pallas as pl
from jax.experimental.pallas import tpu as pltpu
from jax.experimental.pallas import tpu_sc as plsc

_NUM_CORES = 2
_NUM_SUBCORES = 16
_NUM_WORKERS = _NUM_CORES * _NUM_SUBCORES
_IDX_PER_STREAM = 128


@functools.cache
def _make_gather(n_idx: int, vocab: int, dim: int):
  per_worker = n_idx // _NUM_WORKERS
  assert per_worker * _NUM_WORKERS == n_idx
  chunk = 1024
  while per_worker % chunk:
    chunk //= 2
  n_streams = chunk // _IDX_PER_STREAM
  n_chunks = per_worker // chunk
  assert n_chunks >= 4
  row_align = min(n_streams, 8)

  mesh = plsc.VectorSubcoreMesh(core_axis_name="c", subcore_axis_name="s")

  @functools.partial(
      pl.kernel,
      mesh=mesh,
      out_type=jax.ShapeDtypeStruct((n_idx, dim), jnp.float32),
      scratch_types=[
          pltpu.VMEM((2, n_streams, _IDX_PER_STREAM), jnp.int32),
          pltpu.VMEM((2, chunk, dim), jnp.float32),
          pltpu.SemaphoreType.DMA,
          pltpu.SemaphoreType.DMA,
          pltpu.SemaphoreType.DMA,
      ],
      compiler_params=pltpu.CompilerParams(use_tc_tiling_on_sc=False),
  )
  def gather(idx_hbm, table_hbm, out_hbm, idx_v, rows_v, sem_i, sem_g, sem_w):
    wid = lax.axis_index("s") * _NUM_CORES + lax.axis_index("c")
    base = wid * per_worker
    idx_base = base // _IDX_PER_STREAM

    def fire_idx(i, b):
      row = pl.multiple_of(idx_base + i * n_streams, row_align)
      pltpu.async_copy(idx_hbm.at[pl.ds(row, n_streams)], idx_v.at[b], sem_i)

    def drain_idx(b):
      pltpu.make_async_copy(
          idx_hbm.at[pl.ds(0, n_streams)], idx_v.at[b], sem_i
      ).wait()

    def fire_gathers(b):
      for j in range(n_streams):
        pltpu.async_copy(
            table_hbm.at[idx_v.at[b].at[j]],
            rows_v.at[b].at[pl.ds(j * _IDX_PER_STREAM, _IDX_PER_STREAM)],
            sem_g,
        )

    def drain_gathers(b):
      pltpu.make_async_copy(
          out_hbm.at[pl.ds(0, chunk)], rows_v.at[b], sem_g
      ).wait()

    def fire_write(i, b):
      off = pl.multiple_of(base + i * chunk, chunk)
      pltpu.async_copy(rows_v.at[b], out_hbm.at[pl.ds(off, chunk)], sem_w)

    def drain_write(b):
      pltpu.make_async_copy(
          rows_v.at[b], out_hbm.at[pl.ds(0, chunk)], sem_w
      ).wait()

    fire_idx(0, 0)
    drain_idx(0)
    fire_idx(1, 1)
    fire_gathers(0)
    drain_gathers(0)
    drain_idx(1)
    fire_write(0, 0)
    fire_idx(2, 0)
    fire_gathers(1)

    def body(i, _):
      b = i % 2
      nb = 1 - b
      drain_gathers(nb)
      drain_write(b)
      drain_idx(b)
      fire_write(i - 1, nb)
      fire_idx(i + 1, nb)
      fire_gathers(b)
      return 0

    lax.fori_loop(2, n_chunks - 1, body, 0, unroll=False)

    last = n_chunks - 1
    b = last % 2
    nb = 1 - b
    drain_gathers(nb)
    drain_write(b)
    drain_idx(b)
    fire_write(last - 1, nb)
    fire_gathers(b)
    drain_gathers(b)
    fire_write(last, b)
    drain_write(nb)
    drain_write(b)

  return gather


def kernel(input, lookup):
  batch, hist = input.shape
  vocab, dim = lookup.shape
  n_idx = batch * hist
  idx = input.reshape(n_idx // _IDX_PER_STREAM, _IDX_PER_STREAM).astype(
      jnp.int32
  )
  out = _make_gather(n_idx, vocab, dim)(idx, lookup)
  return out.reshape(batch, hist, dim)


# speedup vs baseline: 1.1105x; 1.0002x over previous
import functools

import jax
import jax.numpy as jnp
from jax import lax
from jax.experimental import pallas as pl
from jax.experimental.pallas import tpu as pltpu
from jax.experimental.pallas import tpu_sc as plsc

_NUM_CORES = 2
_NUM_SUBCORES = 16
_NUM_WORKERS = _NUM_CORES * _NUM_SUBCORES
_IDX_PER_STREAM = 128


@functools.cache
def _make_gather(n_idx: int, vocab: int, dim: int):
  per_worker = n_idx // _NUM_WORKERS
  assert per_worker * _NUM_WORKERS == n_idx
  chunk = 1024
  while per_worker % chunk:
    chunk //= 2
  n_streams = chunk // _IDX_PER_STREAM
  n_chunks = per_worker // chunk
  assert n_chunks >= 4
  row_align = min(n_streams, 8)

  mesh = plsc.VectorSubcoreMesh(core_axis_name="c", subcore_axis_name="s")

  @functools.partial(
      pl.kernel,
      mesh=mesh,
      out_type=jax.ShapeDtypeStruct((n_idx, dim), jnp.float32),
      scratch_types=[
          pltpu.VMEM((2, n_streams, _IDX_PER_STREAM), jnp.int32),
          pltpu.VMEM((2, chunk, dim), jnp.float32),
          pltpu.SemaphoreType.DMA,
          pltpu.SemaphoreType.DMA,
          pltpu.SemaphoreType.DMA,
      ],
      compiler_params=pltpu.CompilerParams(use_tc_tiling_on_sc=False),
  )
  def gather(idx_hbm, table_hbm, out_hbm, idx_v, rows_v, sem_i, sem_g, sem_w):
    wid = lax.axis_index("s") * _NUM_CORES + lax.axis_index("c")
    base = wid * per_worker
    idx_base = base // _IDX_PER_STREAM

    def fire_idx(i, b):
      row = pl.multiple_of(idx_base + i * n_streams, row_align)
      pltpu.async_copy(idx_hbm.at[pl.ds(row, n_streams)], idx_v.at[b], sem_i)

    def drain_idx(b):
      pltpu.make_async_copy(
          idx_hbm.at[pl.ds(0, n_streams)], idx_v.at[b], sem_i
      ).wait()

    sub = 32
    n_sub = _IDX_PER_STREAM // sub

    def fire_gathers(b):
      for j in range(n_streams):
        for k in range(n_sub):
          pltpu.async_copy(
              table_hbm.at[idx_v.at[b].at[j, pl.ds(k * sub, sub)]],
              rows_v.at[b].at[
                  pl.ds(j * _IDX_PER_STREAM + k * sub, sub)
              ],
              sem_g,
          )

    def drain_gathers(b):
      pltpu.make_async_copy(
          out_hbm.at[pl.ds(0, chunk)], rows_v.at[b], sem_g
      ).wait()

    def fire_write(i, b):
      off = pl.multiple_of(base + i * chunk, chunk)
      pltpu.async_copy(rows_v.at[b], out_hbm.at[pl.ds(off, chunk)], sem_w)

    def drain_write(b):
      pltpu.make_async_copy(
          rows_v.at[b], out_hbm.at[pl.ds(0, chunk)], sem_w
      ).wait()

    fire_idx(0, 0)
    drain_idx(0)
    fire_idx(1, 1)
    fire_gathers(0)
    drain_gathers(0)
    drain_idx(1)
    fire_write(0, 0)
    fire_idx(2, 0)
    fire_gathers(1)

    def body(i, _):
      b = i % 2
      nb = 1 - b
      drain_gathers(nb)
      drain_write(b)
      drain_idx(b)
      fire_write(i - 1, nb)
      fire_idx(i + 1, nb)
      fire_gathers(b)
      return 0

    lax.fori_loop(2, n_chunks - 1, body, 0, unroll=False)

    last = n_chunks - 1
    b = last % 2
    nb = 1 - b
    drain_gathers(nb)
    drain_write(b)
    drain_idx(b)
    fire_write(last - 1, nb)
    fire_gathers(b)
    drain_gathers(b)
    fire_write(last, b)
    drain_write(nb)
    drain_write(b)

  return gather


def kernel(input, lookup):
  batch, hist = input.shape
  vocab, dim = lookup.shape
  n_idx = batch * hist
  idx = input.reshape(n_idx // _IDX_PER_STREAM, _IDX_PER_STREAM).astype(
      jnp.int32
  )
  out = _make_gather(n_idx, vocab, dim)(idx, lookup)
  return out.reshape(batch, hist, dim)


# speedup vs baseline: 1.7011x; 1.5319x over previous
import functools

import jax
import jax.numpy as jnp
from jax import lax
from jax.experimental import pallas as pl
from jax.experimental.pallas import tpu as pltpu
from jax.experimental.pallas import tpu_sc as plsc

_NC = 2
_NS = 16
_NW = _NC * _NS
_LANES = 128
_VL = 16
_NB = 4
_SLAB = 8


@functools.cache
def _make_lookup(batch: int, hist: int, vocab: int, dim: int):
  n_blocks = batch // (_NW * _NB)
  n_slabs = n_blocks // 2
  assert n_blocks * _NW * _NB == batch and n_slabs * 2 == n_blocks
  assert n_slabs >= 3

  mesh = plsc.VectorSubcoreMesh(core_axis_name="c", subcore_axis_name="s")

  @functools.partial(
      pl.kernel,
      mesh=mesh,
      out_type=jax.ShapeDtypeStruct((batch, hist, dim), jnp.float32),
      scratch_types=[
          pltpu.VMEM((2, _SLAB, hist), jnp.int32),
          pltpu.VMEM((_NB, hist, _LANES), jnp.float32),
          pltpu.VMEM((_NB, hist, _LANES), jnp.float32),
          pltpu.VMEM((_NB, hist, dim), jnp.float32),
          pltpu.VMEM((_NB, hist, dim), jnp.float32),
          pltpu.SemaphoreType.DMA,
          pltpu.SemaphoreType.DMA,
          pltpu.SemaphoreType.DMA,
          pltpu.SemaphoreType.DMA,
      ],
  )
  def body(
      idx_hbm,
      table_hbm,
      out_hbm,
      idx_v,
      rows0,
      rows1,
      comp0,
      comp1,
      sem_i,
      sem_g,
      sem_o0,
      sem_o1,
  ):
    wid = lax.axis_index("s") * _NC + lax.axis_index("c")
    base = wid * n_blocks * _NB

    def fire_idx(s, sb):
      off = pl.multiple_of(base + s * _SLAB, _SLAB)
      pltpu.async_copy(idx_hbm.at[pl.ds(off, _SLAB)], idx_v.at[sb], sem_i)

    def drain_idx(sb):
      pltpu.make_async_copy(
          idx_hbm.at[pl.ds(0, _SLAB)], idx_v.at[sb], sem_i
      ).wait()

    def fire_gathers(rows, sb, half):
      for r in range(_NB):
        pltpu.async_copy(
            table_hbm.at[idx_v.at[sb].at[half * _NB + r]],
            rows.at[r],
            sem_g,
        )

    def drain_gathers(rows, sb, half):
      for r in range(_NB):
        pltpu.make_async_copy(
            table_hbm.at[idx_v.at[sb].at[half * _NB + r]],
            rows.at[r],
            sem_g,
        ).wait()

    def repack(rows, comp):
      def per_r(r, _):
        for h in range(hist):
          for v in range(dim // _VL):
            comp[r, h, pl.ds(v * _VL, _VL)] = rows[r, h, pl.ds(v * _VL, _VL)]
        return 0

      lax.fori_loop(0, _NB, per_r, 0, unroll=False)

    def fire_write(j, comp, sem):
      off = base + j * _NB
      pltpu.async_copy(comp, out_hbm.at[pl.ds(off, _NB)], sem)

    def drain_write(comp, sem):
      pltpu.make_async_copy(comp, out_hbm.at[pl.ds(0, _NB)], sem).wait()

    fire_idx(0, 0)
    drain_idx(0)
    fire_gathers(rows0, 0, 0)
    fire_idx(1, 1)
    drain_gathers(rows0, 0, 0)
    fire_gathers(rows1, 0, 1)
    repack(rows0, comp0)
    fire_write(0, comp0, sem_o0)
    drain_idx(1)
    drain_gathers(rows1, 0, 1)
    fire_gathers(rows0, 1, 0)
    repack(rows1, comp1)
    fire_write(1, comp1, sem_o1)
    drain_gathers(rows0, 1, 0)
    fire_gathers(rows1, 1, 1)
    fire_idx(2, 0)
    drain_write(comp0, sem_o0)
    repack(rows0, comp0)
    fire_write(2, comp0, sem_o0)

    def step(s, _):
      sb = s % 2
      drain_idx(sb)
      drain_gathers(rows1, sb, 1)
      fire_gathers(rows0, sb, 0)
      drain_write(comp1, sem_o1)
      repack(rows1, comp1)
      fire_write(2 * s - 1, comp1, sem_o1)
      drain_gathers(rows0, sb, 0)
      fire_gathers(rows1, sb, 1)
      fire_idx(s + 1, 1 - sb)
      drain_write(comp0, sem_o0)
      repack(rows0, comp0)
      fire_write(2 * s, comp0, sem_o0)
      return 0

    lax.fori_loop(2, n_slabs - 1, step, 0, unroll=False)

    s = n_slabs - 1
    sb = s % 2
    drain_idx(sb)
    drain_gathers(rows1, sb, 1)
    fire_gathers(rows0, sb, 0)
    drain_write(comp1, sem_o1)
    repack(rows1, comp1)
    fire_write(2 * s - 1, comp1, sem_o1)
    drain_gathers(rows0, sb, 0)
    fire_gathers(rows1, sb, 1)
    drain_write(comp0, sem_o0)
    repack(rows0, comp0)
    fire_write(2 * s, comp0, sem_o0)
    drain_gathers(rows1, sb, 1)
    drain_write(comp1, sem_o1)
    repack(rows1, comp1)
    fire_write(2 * s + 1, comp1, sem_o1)
    drain_write(comp0, sem_o0)
    drain_write(comp1, sem_o1)

  return body


def kernel(input, lookup):
  batch, hist = input.shape
  vocab, dim = lookup.shape
  table = jnp.pad(lookup, ((0, 0), (0, _LANES - dim)))
  return _make_lookup(batch, hist, vocab, dim)(input, table)
